# Initial kernel scaffold; baseline (speedup 1.0000x reference)
import jax
import jax.numpy as jnp
from jax.experimental import pallas as pl


def kernel(latent_features, node_features, edge_features, rigids_t, node_mask, params, edge_index):
    raise NotImplementedError("write your pallas kernel here")


# speedup vs baseline: 5.1169x; 5.1169x over previous
import functools

import jax
import jax.numpy as jnp
from jax import lax
from jax.experimental import pallas as pl
from jax.experimental.pallas import tpu as pltpu
from jax.experimental.pallas import tpu_sc as plsc

N = 10000
E = 160000
K = 16
NUM_LAYERS = 4
F = 128
NC, NS = 2, 16
NW = NC * NS
CH = 128
NCHUNK = E // CH
ITERS = -(-NCHUNK // NW)
NPAD = 10240
RPT = NPAD // NS
ZR = 40

_MESH = plsc.VectorSubcoreMesh(core_axis_name="c", subcore_axis_name="s",
                               num_cores=NC, num_subcores=NS)


def _make_gather(d):
  def body(p_hbm, q_hbm, src_hbm, dst_hbm, gs_hbm, gd_hbm, idx_v, rows_v, sem):
    wid = lax.axis_index("s") * NC + lax.axis_index("c")

    def step(i, carry):
      c = i * NW + wid

      @pl.when(c < NCHUNK)
      def _():
        off = pl.multiple_of(c * CH, 8)
        pltpu.sync_copy(src_hbm.at[pl.ds(off, CH)], idx_v)
        pltpu.async_copy(p_hbm.at[idx_v], rows_v, sem).wait()
        pltpu.sync_copy(rows_v, gs_hbm.at[pl.ds(off, CH)])
        pltpu.sync_copy(dst_hbm.at[pl.ds(off, CH)], idx_v)
        pltpu.async_copy(q_hbm.at[idx_v], rows_v, sem).wait()
        pltpu.sync_copy(rows_v, gd_hbm.at[pl.ds(off, CH)])

      return carry

    lax.fori_loop(0, ITERS, step, 0)

  return pl.kernel(
      body,
      out_type=[jax.ShapeDtypeStruct((E, d), jnp.float32)] * 2,
      mesh=_MESH,
      compiler_params=pltpu.CompilerParams(use_tc_tiling_on_sc=(d == F)),
      scratch_types=[
          pltpu.VMEM((CH,), jnp.int32),
          pltpu.VMEM((CH, d), jnp.float32),
          pltpu.SemaphoreType.DMA,
      ],
  )


_gather_f = _make_gather(F)
_gather_g = _make_gather(16)


def _make_scatter(d):
  def body(v_hbm, dst2_hbm, out_hbm, idx_v, rows_v, zbuf, s_sh, sem):
    cid = lax.axis_index("c")
    sid = lax.axis_index("s")
    wid = sid * NC + cid

    for r in range(ZR):
      for col in range(d // 16):
        zbuf[r, pl.ds(col * 16, 16)] = jnp.zeros((16,), jnp.float32)
    for i in range(RPT // ZR):
      pltpu.sync_copy(zbuf, s_sh.at[pl.ds(sid * RPT + i * ZR, ZR)])
    plsc.subcore_barrier()

    def step(i, carry):
      c = i * NW + wid

      @pl.when(c < NCHUNK)
      def _():
        off = pl.multiple_of(c * CH, 8)
        pltpu.sync_copy(v_hbm.at[pl.ds(off, CH)], rows_v)
        pltpu.sync_copy(dst2_hbm.at[pl.ds(c, 1)], idx_v)
        pltpu.sync_copy(rows_v, s_sh.at[idx_v.at[0]], add=True)

      return carry

    lax.fori_loop(0, ITERS, step, 0)
    plsc.subcore_barrier()
    pltpu.sync_copy(
        s_sh.at[pl.ds(sid * RPT, RPT)],
        out_hbm.at[pl.ds(cid * NPAD + sid * RPT, RPT)],
    )

  return pl.kernel(
      body,
      out_type=jax.ShapeDtypeStruct((NC * NPAD, d), jnp.float32),
      mesh=_MESH,
      compiler_params=pltpu.CompilerParams(use_tc_tiling_on_sc=(d == F)),
      scratch_types=[
          pltpu.VMEM((1, CH), jnp.int32),
          pltpu.VMEM((CH, d), jnp.float32),
          pltpu.VMEM((ZR, d), jnp.float32),
          pltpu.MemorySpace.VMEM_SHARED((NPAD, d), jnp.float32),
          pltpu.SemaphoreType.DMA,
      ],
  )


_scatter_f = _make_scatter(F)
_scatter_g = _make_scatter(16)


_BE = 2000


def _geo_body(ts_ref, td_ref, crel_ref, cm_ref, dist_ref, em_ref, em16_ref):
  ts = ts_ref[...]
  td = td_ref[...]
  diff = ts - td
  d2 = jnp.sum(diff * diff * crel_ref[...], axis=1, keepdims=True)
  dist_ref[...] = jnp.sqrt(d2 + 1e-8)
  am = jnp.sum(ts * cm_ref[...], axis=1, keepdims=True)
  bm = jnp.sum(td * cm_ref[...], axis=1, keepdims=True)
  em = am * bm
  em_ref[...] = em
  em16_ref[...] = jnp.broadcast_to(em, em16_ref.shape)


_geo_call = pl.pallas_call(
    _geo_body,
    grid=(E // _BE,),
    in_specs=[
        pl.BlockSpec((_BE, 16), lambda i: (i, 0)),
        pl.BlockSpec((_BE, 16), lambda i: (i, 0)),
        pl.BlockSpec((1, 16), lambda i: (0, 0)),
        pl.BlockSpec((1, 16), lambda i: (0, 0)),
    ],
    out_specs=[
        pl.BlockSpec((_BE, 1), lambda i: (i, 0)),
        pl.BlockSpec((_BE, 1), lambda i: (i, 0)),
        pl.BlockSpec((_BE, 16), lambda i: (i, 0)),
    ],
    out_shape=[
        jax.ShapeDtypeStruct((E, 1), jnp.float32),
        jax.ShapeDtypeStruct((E, 1), jnp.float32),
        jax.ShapeDtypeStruct((E, 16), jnp.float32),
    ],
)


_BN = 2000


def _pq_body(node_ref, lat_ref, r_ref, a1a, a1b, a2a, a2b, ag3, b1, p_ref, q_ref):
  node = node_ref[...]
  latv = lat_ref[...]
  rg = jnp.dot(r_ref[...], ag3[...], preferred_element_type=jnp.float32)
  p = (jnp.dot(node, a1a[...], preferred_element_type=jnp.float32)
       + jnp.dot(latv, a1b[...], preferred_element_type=jnp.float32))
  q = (jnp.dot(node, a2a[...], preferred_element_type=jnp.float32)
       + jnp.dot(latv, a2b[...], preferred_element_type=jnp.float32))
  p_ref[...] = p + rg + b1[...]
  q_ref[...] = q - rg


_pq_call = pl.pallas_call(
    _pq_body,
    grid=(N // _BN,),
    in_specs=[
        pl.BlockSpec((_BN, F), lambda i: (i, 0)),
        pl.BlockSpec((_BN, F), lambda i: (i, 0)),
        pl.BlockSpec((_BN, 3), lambda i: (i, 0)),
        pl.BlockSpec((F, F), lambda i: (0, 0)),
        pl.BlockSpec((F, F), lambda i: (0, 0)),
        pl.BlockSpec((F, F), lambda i: (0, 0)),
        pl.BlockSpec((F, F), lambda i: (0, 0)),
        pl.BlockSpec((3, F), lambda i: (0, 0)),
        pl.BlockSpec((1, F), lambda i: (0, 0)),
    ],
    out_specs=[
        pl.BlockSpec((_BN, F), lambda i: (i, 0)),
        pl.BlockSpec((_BN, F), lambda i: (i, 0)),
    ],
    out_shape=[jax.ShapeDtypeStruct((N, F), jnp.float32)] * 2,
)


def _edge_body(gs_ref, gd_ref, z_ref, dist_ref, em_ref, az, we, ag4, be,
                    zo_ref, hm_ref):
  z = z_ref[...]
  pre = (gs_ref[...] + gd_ref[...]
         + jnp.dot(z, az[...], preferred_element_type=jnp.float32)
         + dist_ref[...] * ag4[...])
  hmid = jnp.maximum(pre, 0.0)
  em = em_ref[...]
  hm_ref[...] = hmid * em
  zo_ref[...] = z + (jnp.dot(hmid, we[...], preferred_element_type=jnp.float32)
                     + be[...]) * em


_edge_call = pl.pallas_call(
    _edge_body,
    grid=(E // _BE,),
    in_specs=[
        pl.BlockSpec((_BE, F), lambda i: (i, 0)),
        pl.BlockSpec((_BE, F), lambda i: (i, 0)),
        pl.BlockSpec((_BE, F), lambda i: (i, 0)),
        pl.BlockSpec((_BE, 1), lambda i: (i, 0)),
        pl.BlockSpec((_BE, 1), lambda i: (i, 0)),
        pl.BlockSpec((F, F), lambda i: (0, 0)),
        pl.BlockSpec((F, F), lambda i: (0, 0)),
        pl.BlockSpec((1, F), lambda i: (0, 0)),
        pl.BlockSpec((1, F), lambda i: (0, 0)),
    ],
    out_specs=[
        pl.BlockSpec((_BE, F), lambda i: (i, 0)),
        pl.BlockSpec((_BE, F), lambda i: (i, 0)),
    ],
    out_shape=[jax.ShapeDtypeStruct((E, F), jnp.float32)] * 2,
)


def _node_body(node_ref, lat_ref, s0_ref, s1_ref, d0_ref, d1_ref, mask_ref,
               w2, b2k, wlat, wnode, gamma, beta, lo_ref, no_ref):
  node = node_ref[...]
  latv = lat_ref[...]
  s = s0_ref[...] + s1_ref[...]
  deg = jnp.sum(d0_ref[...] + d1_ref[...], axis=1, keepdims=True) * (1.0 / 16.0)
  agg = (jnp.dot(s, w2[...], preferred_element_type=jnp.float32) * (1.0 / K)
         + deg * b2k[...])
  mask = mask_ref[...]
  jn = (node + agg[:, 0:F]) * mask
  jl = (latv + agg[:, F:2 * F]) * mask
  jz = agg[:, 2 * F:3 * F] * mask
  joint = jnp.concatenate([jn, jl, jz], axis=1)
  lo_ref[...] = latv + jnp.dot(joint, wlat[...], preferred_element_type=jnp.float32)
  npre = node + jnp.dot(joint, wnode[...], preferred_element_type=jnp.float32)
  mu = jnp.mean(npre, axis=1, keepdims=True)
  var = jnp.mean((npre - mu) ** 2, axis=1, keepdims=True)
  no_ref[...] = (npre - mu) / jnp.sqrt(var + 1e-5) * gamma[...] + beta[...]


_node_call = pl.pallas_call(
    _node_body,
    grid=(N // _BN,),
    in_specs=[
        pl.BlockSpec((_BN, F), lambda i: (i, 0)),
        pl.BlockSpec((_BN, F), lambda i: (i, 0)),
        pl.BlockSpec((_BN, F), lambda i: (i, 0)),
        pl.BlockSpec((_BN, F), lambda i: (i, 0)),
        pl.BlockSpec((_BN, 16), lambda i: (i, 0)),
        pl.BlockSpec((_BN, 16), lambda i: (i, 0)),
        pl.BlockSpec((_BN, 1), lambda i: (i, 0)),
        pl.BlockSpec((F, 3 * F), lambda i: (0, 0)),
        pl.BlockSpec((1, 3 * F), lambda i: (0, 0)),
        pl.BlockSpec((3 * F, F), lambda i: (0, 0)),
        pl.BlockSpec((3 * F, F), lambda i: (0, 0)),
        pl.BlockSpec((1, F), lambda i: (0, 0)),
        pl.BlockSpec((1, F), lambda i: (0, 0)),
    ],
    out_specs=[
        pl.BlockSpec((_BN, F), lambda i: (i, 0)),
        pl.BlockSpec((_BN, F), lambda i: (i, 0)),
    ],
    out_shape=[jax.ShapeDtypeStruct((N, F), jnp.float32)] * 2,
)


def kernel(latent_features, node_features, edge_features, rigids_t, node_mask,
           params, edge_index):
  src = edge_index[1]
  dst = edge_index[0]
  dst2 = dst.reshape(NCHUNK, CH)

  tm = jnp.concatenate(
      [rigids_t, node_mask[:, None], jnp.zeros((N, 12), jnp.float32)], axis=1)
  ts, td = _gather_g(tm, tm, src, dst)
  crel = jnp.concatenate(
      [jnp.ones((1, 3), jnp.float32), jnp.zeros((1, 13), jnp.float32)], axis=1)
  cm = jnp.concatenate(
      [jnp.zeros((1, 3), jnp.float32), jnp.ones((1, 1), jnp.float32),
       jnp.zeros((1, 12), jnp.float32)], axis=1)
  dist, em, em16 = _geo_call(ts, td, crel, cm)
  deg16 = _scatter_g(em16, dst2)

  node = node_features
  lat = latent_features
  z = edge_features
  maskc = node_mask[:, None]

  for l in range(NUM_LAYERS):
    W1 = params['W1'][l]
    a1a, a1b = W1[0:128], W1[128:256]
    a2a, a2b = W1[384:512], W1[512:640]
    az, ag3, ag4 = W1[768:896], W1[896:899], W1[899:900]
    b1 = params['b1'][l][None]
    p, q = _pq_call(node, lat, rigids_t, a1a, a1b, a2a, a2b, ag3, b1)
    gs, gd = _gather_f(p, q, src, dst)
    z, hm = _edge_call(gs, gd, z, dist, em, az, params['We'][l],
                       ag4, params['be'][l][None])
    s = _scatter_f(hm, dst2)
    b2k = (params['b2'][l] / K)[None]
    lat, node = _node_call(node, lat, s[0:N], s[NPAD:NPAD + N],
                           deg16[0:N], deg16[NPAD:NPAD + N], maskc,
                           params['W2'][l], b2k, params['Wlat'][l],
                           params['Wnode'][l], params['gamma'][l][None],
                           params['beta'][l][None])
  return lat
